# Initial kernel scaffold; baseline (speedup 1.0000x reference)
import jax
import jax.numpy as jnp
from jax.experimental import pallas as pl


def kernel(stimulus_set, percept_embeddings):
    raise NotImplementedError("write your pallas kernel here")


# speedup vs baseline: 12.5183x; 12.5183x over previous
import functools

import jax
import jax.numpy as jnp
from jax import lax
from jax.experimental import pallas as pl
from jax.experimental.pallas import tpu as pltpu
from jax.experimental.pallas import tpu_sc as plsc

_TBL = 32
_NTBL = _TBL * _TBL
_EPAD = 16


def _rate_table_body(e_ref, r_ref):
    d2 = jnp.zeros((_TBL, _TBL), jnp.float32)
    for k in range(_EPAD):
        col = e_ref[:, k]
        diff = col[:, None] - col[None, :]
        d2 += diff * diff
    d = jnp.sqrt(d2)
    s = jnp.exp(-3.0 * d)
    r_ref[...] = 1.0 / (1.0 + jnp.exp(-5.0 * (s - 0.5)))


def _make_sc_gather(n_pairs):
    info = plsc.get_sparse_core_info()
    nw = info.num_cores * info.num_subcores
    per_tile = n_pairs // nw
    assert per_tile * nw == n_pairs
    chunk = 4096
    while per_tile % chunk:
        chunk //= 2
    nch = per_tile // chunk

    mesh = plsc.VectorSubcoreMesh(core_axis_name="c", subcore_axis_name="s")

    @functools.partial(
        pl.kernel,
        mesh=mesh,
        out_type=jax.ShapeDtypeStruct((n_pairs,), jnp.float32),
        compiler_params=pltpu.CompilerParams(needs_layout_passes=False),
        scratch_types=[
            pltpu.VMEM((_NTBL,), jnp.float32),
            pltpu.VMEM((2 * chunk + 16,), jnp.int32),
            pltpu.VMEM((chunk + 8,), jnp.float32),
        ],
    )
    def sc_gather(r_hbm, idx_hbm, out_hbm, r_v, in_v, out_v):
        wid = lax.axis_index("s") * info.num_cores + lax.axis_index("c")
        base = wid * per_tile
        pltpu.sync_copy(r_hbm, r_v)
        even = (lax.iota(jnp.int32, 16) & 1) == 0

        def chunk_body(c, _):
            in_off = (base + c * chunk) * 2
            pltpu.sync_copy(idx_hbm.at[pl.ds(in_off, 2 * chunk)],
                            in_v.at[pl.ds(0, 2 * chunk)])

            def vec_body(k, _):
                v = in_v[pl.ds(16 * k, 16)]
                vs = in_v[pl.ds(16 * k + 1, 16)]
                comb = ((v << 5) + vs) & (_NTBL - 1)
                r = plsc.load_gather(r_v, [comb])
                plsc.store_compressed(out_v.at[pl.ds(8 * k, 16)], r, mask=even)
                return 0

            lax.fori_loop(0, chunk // 8, vec_body, 0, unroll=8)
            pltpu.sync_copy(out_v.at[pl.ds(0, chunk)],
                            out_hbm.at[pl.ds(base + c * chunk, chunk)])
            return 0

        lax.fori_loop(0, nch, chunk_body, 0)

    return sc_gather


def kernel(stimulus_set, percept_embeddings):
    b, t, two = stimulus_set.shape
    n_pairs = b * t
    e_pad = jnp.zeros((_TBL, _EPAD), jnp.float32)
    e_pad = e_pad.at[: percept_embeddings.shape[0], : percept_embeddings.shape[1]].set(
        percept_embeddings)
    r2d = pl.pallas_call(
        _rate_table_body,
        out_shape=jax.ShapeDtypeStruct((_TBL, _TBL), jnp.float32),
    )(e_pad)
    r_flat = r2d.reshape(_NTBL)
    idx_flat = stimulus_set.reshape(n_pairs * two)
    out = _make_sc_gather(n_pairs)(r_flat, idx_flat)
    return out.reshape(b, t, 1)


# speedup vs baseline: 398.5504x; 31.8373x over previous
import functools

import jax
import jax.numpy as jnp
from jax import lax
from jax.experimental import pallas as pl
from jax.experimental.pallas import tpu as pltpu
from jax.experimental.pallas import tpu_sc as plsc

_TBL = 32
_NTBL = _TBL * _TBL
_EPAD = 16


def _rate_table_body(e_ref, r_ref):
    d2 = jnp.zeros((_TBL, _TBL), jnp.float32)
    for k in range(_EPAD):
        col = e_ref[:, k]
        diff = col[:, None] - col[None, :]
        d2 += diff * diff
    d = jnp.sqrt(d2)
    s = jnp.exp(-3.0 * d)
    r_ref[...] = 1.0 / (1.0 + jnp.exp(-5.0 * (s - 0.5)))


def _make_sc_gather(n_t, n_b):
    info = plsc.get_sparse_core_info()
    nw = info.num_cores * info.num_subcores
    n_pairs = n_t * n_b
    qb = n_b // 4
    n_units = n_pairs // qb
    per_tile = n_units // nw
    assert per_tile * nw == n_units and qb % 128 == 0

    mesh = plsc.VectorSubcoreMesh(core_axis_name="c", subcore_axis_name="s")

    @functools.partial(
        pl.kernel,
        mesh=mesh,
        out_type=jax.ShapeDtypeStruct((n_t, 1, n_b), jnp.float32),
        compiler_params=pltpu.CompilerParams(needs_layout_passes=False),
        scratch_types=[
            pltpu.VMEM((_NTBL,), jnp.float32),
            pltpu.VMEM((qb,), jnp.int32),
            pltpu.VMEM((qb,), jnp.int32),
            pltpu.VMEM((qb,), jnp.float32),
        ],
    )
    def sc_gather(r_hbm, x_hbm, out_hbm, r_v, in0_v, in1_v, out_v):
        wid = lax.axis_index("s") * info.num_cores + lax.axis_index("c")
        pltpu.sync_copy(r_hbm, r_v)

        def unit_body(i, _):
            u = wid * per_tile + i
            t = u >> 2
            q = u & 3
            pltpu.sync_copy(x_hbm.at[t, 0, pl.ds(q * qb, qb)], in0_v)
            pltpu.sync_copy(x_hbm.at[t, 1, pl.ds(q * qb, qb)], in1_v)

            def vec_body(j, _):
                v0 = in0_v[pl.ds(j * 16, 16)]
                v1 = in1_v[pl.ds(j * 16, 16)]
                comb = ((v0 << 5) + v1) & (_NTBL - 1)
                out_v[pl.ds(j * 16, 16)] = plsc.load_gather(r_v, [comb])
                return 0

            lax.fori_loop(0, qb // 16, vec_body, 0, unroll=8)
            pltpu.sync_copy(out_v, out_hbm.at[t, 0, pl.ds(q * qb, qb)])
            return 0

        lax.fori_loop(0, per_tile, unit_body, 0)

    return sc_gather


def kernel(stimulus_set, percept_embeddings):
    b, t, two = stimulus_set.shape
    e_pad = jnp.zeros((_TBL, _EPAD), jnp.float32)
    e_pad = e_pad.at[: percept_embeddings.shape[0], : percept_embeddings.shape[1]].set(
        percept_embeddings)
    r2d = pl.pallas_call(
        _rate_table_body,
        out_shape=jax.ShapeDtypeStruct((_TBL, _TBL), jnp.float32),
    )(e_pad)
    r_flat = r2d.reshape(_NTBL)
    x3 = jnp.transpose(stimulus_set, (1, 2, 0))
    out = _make_sc_gather(t, b)(r_flat, x3)
    return jnp.transpose(out, (2, 0, 1))


# speedup vs baseline: 563.9413x; 1.4150x over previous
import functools

import jax
import jax.numpy as jnp
from jax import lax
from jax.experimental import pallas as pl
from jax.experimental.pallas import tpu as pltpu
from jax.experimental.pallas import tpu_sc as plsc

_TBL = 32
_NTBL = _TBL * _TBL
_EPAD = 16


def _rate_table_body(e_ref, r_ref):
    d2 = jnp.zeros((_TBL, _TBL), jnp.float32)
    for k in range(_EPAD):
        col = e_ref[:, k]
        diff = col[:, None] - col[None, :]
        d2 += diff * diff
    d = jnp.sqrt(d2)
    s = jnp.exp(-3.0 * d)
    r_ref[...] = 1.0 / (1.0 + jnp.exp(-5.0 * (s - 0.5)))


def _make_sc_gather(n_t, n_b):
    info = plsc.get_sparse_core_info()
    nw = info.num_cores * info.num_subcores
    n_pairs = n_t * n_b
    qb = n_b // 4
    n_units = n_pairs // qb
    per_tile = n_units // nw
    assert per_tile * nw == n_units and qb % 128 == 0

    mesh = plsc.VectorSubcoreMesh(core_axis_name="c", subcore_axis_name="s")

    nbuf = 4
    scratch = [pltpu.VMEM((_NTBL,), jnp.float32)]
    scratch += [pltpu.VMEM((2, qb), jnp.int32) for _ in range(nbuf)]
    scratch += [pltpu.VMEM((qb,), jnp.float32) for _ in range(nbuf)]
    scratch += [pltpu.SemaphoreType.DMA for _ in range(2 * nbuf)]

    @functools.partial(
        pl.kernel,
        mesh=mesh,
        out_type=jax.ShapeDtypeStruct((n_t, 1, n_b), jnp.float32),
        compiler_params=pltpu.CompilerParams(needs_layout_passes=False),
        scratch_types=scratch,
    )
    def sc_gather(r_hbm, x_hbm, out_hbm, r_v, *bufs):
        in_v = bufs[:nbuf]
        out_v = bufs[nbuf:2 * nbuf]
        in_sem = bufs[2 * nbuf:3 * nbuf]
        out_sem = bufs[3 * nbuf:4 * nbuf]
        wid = lax.axis_index("s") * info.num_cores + lax.axis_index("c")
        u0 = wid * per_tile
        pltpu.sync_copy(r_hbm, r_v)

        def start_in(i):
            u = u0 + i
            return pltpu.async_copy(
                x_hbm.at[u >> 2, :, pl.ds((u & 3) * qb, qb)],
                in_v[i % nbuf], in_sem[i % nbuf])

        in_h = {}
        out_h = {}
        for i in range(min(nbuf, per_tile)):
            in_h[i] = start_in(i)
        for i in range(per_tile):
            b = i % nbuf
            in_h.pop(i).wait()
            if i >= nbuf:
                out_h.pop(i - nbuf).wait()

            def vec_body(j, _, b=b):
                v0 = in_v[b][0, pl.ds(j * 16, 16)]
                v1 = in_v[b][1, pl.ds(j * 16, 16)]
                comb = ((v0 << 5) + v1) & (_NTBL - 1)
                out_v[b][pl.ds(j * 16, 16)] = plsc.load_gather(r_v, [comb])
                return 0

            lax.fori_loop(0, qb // 16, vec_body, 0, unroll=8)
            u = u0 + i
            out_h[i] = pltpu.async_copy(
                out_v[b], out_hbm.at[u >> 2, 0, pl.ds((u & 3) * qb, qb)],
                out_sem[b])
            if i + nbuf < per_tile:
                in_h[i + nbuf] = start_in(i + nbuf)
        for i in sorted(out_h):
            out_h.pop(i).wait()

    return sc_gather


def kernel(stimulus_set, percept_embeddings):
    b, t, two = stimulus_set.shape
    e_pad = jnp.zeros((_TBL, _EPAD), jnp.float32)
    e_pad = e_pad.at[: percept_embeddings.shape[0], : percept_embeddings.shape[1]].set(
        percept_embeddings)
    r2d = pl.pallas_call(
        _rate_table_body,
        out_shape=jax.ShapeDtypeStruct((_TBL, _TBL), jnp.float32),
    )(e_pad)
    r_flat = r2d.reshape(_NTBL)
    x3 = jnp.transpose(stimulus_set, (1, 2, 0))
    out = _make_sc_gather(t, b)(r_flat, x3)
    return jnp.transpose(out, (2, 0, 1))


# speedup vs baseline: 1077.5821x; 1.9108x over previous
import functools

import jax
import jax.numpy as jnp
from jax import lax
from jax.experimental import pallas as pl
from jax.experimental.pallas import tpu as pltpu
from jax.experimental.pallas import tpu_sc as plsc

_TBL = 32
_NTBL = _TBL * _TBL


def _build_rating_table(e_v, r_v, n_stim, n_dim):
    def vec_body(v, _):
        ent = lax.iota(jnp.int32, 16) + v * 16
        i0 = jnp.minimum(ent >> 5, n_stim - 1)
        i1 = jnp.minimum(ent & (_TBL - 1), n_stim - 1)
        acc = jnp.zeros((16,), jnp.float32)
        for d in range(n_dim):
            dcol = jnp.full((16,), d, jnp.int32)
            z0 = plsc.load_gather(e_v, [i0, dcol])
            z1 = plsc.load_gather(e_v, [i1, dcol])
            df = z0 - z1
            acc = acc + df * df
        y = plsc.bitcast((plsc.bitcast(acc, jnp.int32) >> 1) + 0x1FBD1DF5,
                         jnp.float32)
        for _ in range(3):
            y = 0.5 * (y + acc / y)
        s = jnp.exp(-3.0 * y)
        r = 1.0 / (1.0 + jnp.exp(-5.0 * (s - 0.5)))
        r_v[pl.ds(v * 16, 16)] = r
        return 0

    lax.fori_loop(0, _NTBL // 16, vec_body, 0)


def _make_sc_kernel(n_t, n_b, n_stim, n_dim):
    info = plsc.get_sparse_core_info()
    nw = info.num_cores * info.num_subcores
    n_pairs = n_t * n_b
    qb = n_b // 4
    n_units = n_pairs // qb
    per_tile = n_units // nw
    assert per_tile * nw == n_units and qb % 128 == 0

    mesh = plsc.VectorSubcoreMesh(core_axis_name="c", subcore_axis_name="s")

    nbuf = 4
    scratch = [pltpu.VMEM((n_stim, n_dim), jnp.float32),
               pltpu.VMEM((_NTBL,), jnp.float32)]
    scratch += [pltpu.VMEM((2, qb), jnp.int32) for _ in range(nbuf)]
    scratch += [pltpu.VMEM((qb,), jnp.float32) for _ in range(nbuf)]
    scratch += [pltpu.SemaphoreType.DMA for _ in range(2 * nbuf)]

    @functools.partial(
        pl.kernel,
        mesh=mesh,
        out_type=jax.ShapeDtypeStruct((n_t, 1, n_b), jnp.float32),
        compiler_params=pltpu.CompilerParams(needs_layout_passes=False),
        scratch_types=scratch,
    )
    def sc_kernel(e_hbm, x_hbm, out_hbm, e_v, r_v, *bufs):
        in_v = bufs[:nbuf]
        out_v = bufs[nbuf:2 * nbuf]
        in_sem = bufs[2 * nbuf:3 * nbuf]
        out_sem = bufs[3 * nbuf:4 * nbuf]
        wid = lax.axis_index("s") * info.num_cores + lax.axis_index("c")
        u0 = wid * per_tile

        def start_in(i):
            u = u0 + i
            return pltpu.async_copy(
                x_hbm.at[u >> 2, :, pl.ds((u & 3) * qb, qb)],
                in_v[i % nbuf], in_sem[i % nbuf])

        in_h = {}
        out_h = {}
        for i in range(min(nbuf, per_tile)):
            in_h[i] = start_in(i)

        pltpu.sync_copy(e_hbm, e_v)
        _build_rating_table(e_v, r_v, n_stim, n_dim)

        for i in range(per_tile):
            b = i % nbuf
            in_h.pop(i).wait()
            if i >= nbuf:
                out_h.pop(i - nbuf).wait()
            iv = in_v[b]
            ov = out_v[b]

            @plsc.parallel_loop(0, qb, 16, unroll=8)
            def vec_body(j):
                v0 = iv[0, pl.ds(j, 16)]
                v1 = iv[1, pl.ds(j, 16)]
                comb = ((v0 << 5) + v1) & (_NTBL - 1)
                ov[pl.ds(j, 16)] = plsc.load_gather(r_v, [comb])

            u = u0 + i
            out_h[i] = pltpu.async_copy(
                ov, out_hbm.at[u >> 2, 0, pl.ds((u & 3) * qb, qb)],
                out_sem[b])
            if i + nbuf < per_tile:
                in_h[i + nbuf] = start_in(i + nbuf)
        for i in sorted(out_h):
            out_h.pop(i).wait()

    return sc_kernel


def kernel(stimulus_set, percept_embeddings):
    b, t, two = stimulus_set.shape
    n_stim, n_dim = percept_embeddings.shape
    x3 = jnp.transpose(stimulus_set, (1, 2, 0))
    out = _make_sc_kernel(t, b, n_stim, n_dim)(percept_embeddings, x3)
    return jnp.transpose(out, (2, 0, 1))
